# Initial kernel scaffold; baseline (speedup 1.0000x reference)
import jax
import jax.numpy as jnp
from jax.experimental import pallas as pl


def kernel(x, Wr, We):
    raise NotImplementedError("write your pallas kernel here")


# speedup vs baseline: 2.3241x; 2.3241x over previous
import functools

import jax
import jax.numpy as jnp
from jax.experimental import pallas as pl
from jax.experimental.pallas import tpu as pltpu

HIDDEN = 768
N_EXPERTS = 8
BT = 1024


def _moe_body(x_ref, wr_ref, we_ref, out_ref):
    x = x_ref[...]
    wr = wr_ref[...]
    logits = jax.lax.dot_general(
        x, wr, (((1,), (1,)), ((), ())),
        preferred_element_type=jnp.float32)
    mx = jnp.max(logits, axis=1, keepdims=True)
    eids = jax.lax.broadcasted_iota(jnp.int32, logits.shape, 1)
    idx = jnp.min(jnp.where(logits == mx, eids, N_EXPERTS), axis=1)

    acc = jnp.zeros_like(x)
    for e in range(N_EXPERTS):
        ye = jax.lax.dot_general(
            x, we_ref[e], (((1,), (1,)), ((), ())),
            preferred_element_type=jnp.float32)
        m = (idx == e).astype(jnp.float32)[:, None]
        acc = acc + m * ye
    out_ref[...] = acc


@jax.jit
def kernel(x, Wr, We):
    T, H = x.shape
    E = We.shape[0]
    grid = (T // BT,)
    return pl.pallas_call(
        _moe_body,
        grid=grid,
        in_specs=[
            pl.BlockSpec((BT, H), lambda i: (i, 0)),
            pl.BlockSpec((E, H), lambda i: (0, 0)),
            pl.BlockSpec((E, H, H), lambda i: (0, 0, 0)),
        ],
        out_specs=pl.BlockSpec((BT, H), lambda i: (i, 0)),
        out_shape=jax.ShapeDtypeStruct((T, H), jnp.float32),
    )(x, Wr, We)


# speedup vs baseline: 2.3252x; 1.0004x over previous
import functools

import jax
import jax.numpy as jnp
from jax.experimental import pallas as pl
from jax.experimental.pallas import tpu as pltpu

HIDDEN = 768
N_EXPERTS = 8
BT = 1024


def _moe_body(x_ref, wr_ref, we_ref, out_ref):
    x = x_ref[...]
    wr = wr_ref[...]
    logits = jax.lax.dot_general(
        x, wr, (((1,), (1,)), ((), ())),
        preferred_element_type=jnp.float32)
    mx = jnp.max(logits, axis=1, keepdims=True)
    eids = jax.lax.broadcasted_iota(jnp.int32, logits.shape, 1)
    idx = jnp.min(jnp.where(logits == mx, eids, N_EXPERTS), axis=1)

    xb = x.astype(jnp.bfloat16)
    acc = jnp.zeros_like(x)
    for e in range(N_EXPERTS):
        ye = jax.lax.dot_general(
            xb, we_ref[e].astype(jnp.bfloat16), (((1,), (1,)), ((), ())),
            preferred_element_type=jnp.float32)
        m = (idx == e).astype(jnp.float32)[:, None]
        acc = acc + m * ye
    out_ref[...] = acc


@jax.jit
def kernel(x, Wr, We):
    T, H = x.shape
    E = We.shape[0]
    grid = (T // BT,)
    return pl.pallas_call(
        _moe_body,
        grid=grid,
        in_specs=[
            pl.BlockSpec((BT, H), lambda i: (i, 0)),
            pl.BlockSpec((E, H), lambda i: (0, 0)),
            pl.BlockSpec((E, H, H), lambda i: (0, 0, 0)),
        ],
        out_specs=pl.BlockSpec((BT, H), lambda i: (i, 0)),
        out_shape=jax.ShapeDtypeStruct((T, H), jnp.float32),
    )(x, Wr, We)
